# Initial kernel scaffold; baseline (speedup 1.0000x reference)
import jax
import jax.numpy as jnp
from jax.experimental import pallas as pl


def kernel(xyz1, xyz2, feats1, feats2, W, gamma, beta):
    raise NotImplementedError("write your pallas kernel here")


# speedup vs baseline: 30.9455x; 30.9455x over previous
import functools

import jax
import jax.numpy as jnp
from jax.experimental import pallas as pl
from jax.experimental.pallas import tpu as pltpu


def _fprop_kernel(xyz1t_ref, xyz2_ref, f1_ref, f2_ref, wi_ref, wf_ref,
                  y_ref, stats_ref, g2_ref):
    b = pl.program_id(0)
    t = pl.program_id(1)

    @pl.when(jnp.logical_and(b == 0, t == 0))
    def _init_stats():
        stats_ref[...] = jnp.zeros_like(stats_ref)

    @pl.when(t == 0)
    def _compute_g2():
        g2_ref[...] = jnp.dot(wi_ref[...], f2_ref[0],
                              preferred_element_type=jnp.float32)

    at = xyz1t_ref[0]
    bm = xyz2_ref[0]
    a2 = jnp.sum(at * at, axis=0, keepdims=True)
    b2 = jnp.sum(bm * bm, axis=1, keepdims=True)
    d2 = a2 + b2 - 2.0 * jnp.dot(bm, at, preferred_element_type=jnp.float32)
    d = jnp.sqrt(jnp.maximum(d2, 1e-12))

    n2 = d.shape[0]
    rows = jax.lax.broadcasted_iota(jnp.int32, d.shape, 0)
    m = d
    vks = []
    iks = []
    for _ in range(3):
        vk = jnp.min(m, axis=0, keepdims=True)
        cand = jnp.where(m == vk, rows, n2)
        ik = jnp.min(cand, axis=0, keepdims=True)
        vks.append(vk)
        iks.append(ik)
        m = jnp.where(rows == ik, jnp.float32(3.0e38), m)

    w0 = 1.0 / (vks[0] + 1e-8)
    w1 = 1.0 / (vks[1] + 1e-8)
    w2 = 1.0 / (vks[2] + 1e-8)
    ws = w0 + w1 + w2
    w0 = w0 / ws
    w1 = w1 / ws
    w2 = w2 / ws

    st = jnp.where(rows == iks[0], w0, 0.0)
    st = jnp.where(rows == iks[1], w1, st)
    st = jnp.where(rows == iks[2], w2, st)

    y = jnp.dot(g2_ref[...], st, preferred_element_type=jnp.float32)
    y = y + jnp.dot(wf_ref[...], f1_ref[0], preferred_element_type=jnp.float32)
    y_ref[0] = y
    stats_ref[:, 0:1] += jnp.sum(y, axis=1, keepdims=True)
    stats_ref[:, 1:2] += jnp.sum(y * y, axis=1, keepdims=True)


def _bn_kernel(y_ref, stats_ref, gamma_ref, beta_ref, o_ref, *, count):
    s1 = stats_ref[:, 0:1]
    s2 = stats_ref[:, 1:2]
    mean = s1 * (1.0 / count)
    var = s2 * (1.0 / count) - mean * mean
    a = gamma_ref[...] * jax.lax.rsqrt(var + 1e-5)
    c = beta_ref[...] - a * mean
    o_ref[0] = jnp.maximum(y_ref[0] * a + c, 0.0)


def kernel(xyz1, xyz2, feats1, feats2, W, gamma, beta):
    B, N1, _ = xyz1.shape
    N2 = xyz2.shape[1]
    C1 = feats1.shape[1]
    C2 = feats2.shape[1]
    OUT = W.shape[0]
    TN = 512 if N1 % 512 == 0 else N1
    NT = N1 // TN

    xyz1p = jnp.concatenate(
        [xyz1, jnp.zeros((B, N1, 5), xyz1.dtype)], axis=-1)
    xyz1t = jnp.transpose(xyz1p, (0, 2, 1))
    xyz2p = jnp.concatenate(
        [xyz2, jnp.zeros((B, N2, 5), xyz2.dtype)], axis=-1)
    Wi = W[:, :C2]
    Wf = W[:, C2:]

    y, stats = pl.pallas_call(
        _fprop_kernel,
        grid=(B, NT),
        in_specs=[
            pl.BlockSpec((1, 8, TN), lambda b, t: (b, 0, t)),
            pl.BlockSpec((1, N2, 8), lambda b, t: (b, 0, 0)),
            pl.BlockSpec((1, C1, TN), lambda b, t: (b, 0, t)),
            pl.BlockSpec((1, C2, N2), lambda b, t: (b, 0, 0)),
            pl.BlockSpec((OUT, C2), lambda b, t: (0, 0)),
            pl.BlockSpec((OUT, C1), lambda b, t: (0, 0)),
        ],
        out_specs=[
            pl.BlockSpec((1, OUT, TN), lambda b, t: (b, 0, t)),
            pl.BlockSpec((OUT, 2), lambda b, t: (0, 0)),
        ],
        out_shape=[
            jax.ShapeDtypeStruct((B, OUT, N1), jnp.float32),
            jax.ShapeDtypeStruct((OUT, 2), jnp.float32),
        ],
        scratch_shapes=[pltpu.VMEM((OUT, N2), jnp.float32)],
    )(xyz1t, xyz2p, feats1, feats2, Wi, Wf)

    out = pl.pallas_call(
        functools.partial(_bn_kernel, count=float(B * N1)),
        grid=(B, NT),
        in_specs=[
            pl.BlockSpec((1, OUT, TN), lambda b, t: (b, 0, t)),
            pl.BlockSpec((OUT, 2), lambda b, t: (0, 0)),
            pl.BlockSpec((OUT, 1), lambda b, t: (0, 0)),
            pl.BlockSpec((OUT, 1), lambda b, t: (0, 0)),
        ],
        out_specs=pl.BlockSpec((1, OUT, TN), lambda b, t: (b, 0, t)),
        out_shape=jax.ShapeDtypeStruct((B, OUT, N1), jnp.float32),
    )(y, stats, gamma.reshape(OUT, 1), beta.reshape(OUT, 1))
    return out


# speedup vs baseline: 54.9660x; 1.7762x over previous
import functools

import jax
import jax.numpy as jnp
from jax.experimental import pallas as pl
from jax.experimental.pallas import tpu as pltpu


def _fprop_kernel(xyz1a_ref, xyz2a_ref, f1_ref, f2_ref, wi_ref, wf_ref,
                  y_ref, stats_ref, g2_ref):
    b = pl.program_id(0)
    t = pl.program_id(1)

    @pl.when(jnp.logical_and(b == 0, t == 0))
    def _init_stats():
        stats_ref[...] = jnp.zeros_like(stats_ref)

    @pl.when(t == 0)
    def _compute_g2():
        g2_ref[...] = jnp.dot(wi_ref[...], f2_ref[0],
                              preferred_element_type=jnp.float32)

    m = jnp.dot(xyz2a_ref[0], xyz1a_ref[0],
                preferred_element_type=jnp.float32)

    v1 = jnp.min(m, axis=0, keepdims=True)
    m2 = jnp.where(m > v1, m, jnp.float32(3.0e38))
    v2 = jnp.min(m2, axis=0, keepdims=True)
    m3 = jnp.where(m2 > v2, m2, jnp.float32(3.0e38))
    v3 = jnp.min(m3, axis=0, keepdims=True)

    w0 = 1.0 / (jnp.sqrt(jnp.maximum(v1, 1e-12)) + 1e-8)
    w1 = 1.0 / (jnp.sqrt(jnp.maximum(v2, 1e-12)) + 1e-8)
    w2 = 1.0 / (jnp.sqrt(jnp.maximum(v3, 1e-12)) + 1e-8)
    ws = 1.0 / (w0 + w1 + w2)
    w0 = w0 * ws
    w1 = w1 * ws
    w2 = w2 * ws

    st = jnp.where(m == v1, w0, 0.0)
    st = jnp.where(m == v2, w1, st)
    st = jnp.where(m == v3, w2, st)

    y = jnp.dot(g2_ref[...], st, preferred_element_type=jnp.float32)
    y = y + jnp.dot(wf_ref[...], f1_ref[0], preferred_element_type=jnp.float32)
    y_ref[0] = y
    stats_ref[:, 0:1] += jnp.sum(y, axis=1, keepdims=True)
    stats_ref[:, 1:2] += jnp.sum(y * y, axis=1, keepdims=True)


def _bn_kernel(y_ref, stats_ref, gamma_ref, beta_ref, o_ref, *, count):
    s1 = stats_ref[:, 0:1]
    s2 = stats_ref[:, 1:2]
    mean = s1 * (1.0 / count)
    var = s2 * (1.0 / count) - mean * mean
    a = gamma_ref[...] * jax.lax.rsqrt(var + 1e-5)
    c = beta_ref[...] - a * mean
    o_ref[0] = jnp.maximum(y_ref[0] * a + c, 0.0)


def kernel(xyz1, xyz2, feats1, feats2, W, gamma, beta):
    B, N1, _ = xyz1.shape
    N2 = xyz2.shape[1]
    C1 = feats1.shape[1]
    C2 = feats2.shape[1]
    OUT = W.shape[0]
    TN = 512 if N1 % 512 == 0 else N1
    NT = N1 // TN

    ones1 = jnp.ones((B, N1, 1), xyz1.dtype)
    a2 = jnp.sum(xyz1 * xyz1, axis=-1, keepdims=True)
    xyz1a = jnp.concatenate(
        [xyz1, ones1, a2, jnp.zeros((B, N1, 3), xyz1.dtype)], axis=-1)
    xyz1a = jnp.transpose(xyz1a, (0, 2, 1))
    b2 = jnp.sum(xyz2 * xyz2, axis=-1, keepdims=True)
    ones2 = jnp.ones((B, N2, 1), xyz2.dtype)
    xyz2a = jnp.concatenate(
        [-2.0 * xyz2, b2, ones2, jnp.zeros((B, N2, 3), xyz2.dtype)], axis=-1)
    Wi = W[:, :C2]
    Wf = W[:, C2:]

    y, stats = pl.pallas_call(
        _fprop_kernel,
        grid=(B, NT),
        in_specs=[
            pl.BlockSpec((1, 8, TN), lambda b, t: (b, 0, t)),
            pl.BlockSpec((1, N2, 8), lambda b, t: (b, 0, 0)),
            pl.BlockSpec((1, C1, TN), lambda b, t: (b, 0, t)),
            pl.BlockSpec((1, C2, N2), lambda b, t: (b, 0, 0)),
            pl.BlockSpec((OUT, C2), lambda b, t: (0, 0)),
            pl.BlockSpec((OUT, C1), lambda b, t: (0, 0)),
        ],
        out_specs=[
            pl.BlockSpec((1, OUT, TN), lambda b, t: (b, 0, t)),
            pl.BlockSpec((OUT, 2), lambda b, t: (0, 0)),
        ],
        out_shape=[
            jax.ShapeDtypeStruct((B, OUT, N1), jnp.float32),
            jax.ShapeDtypeStruct((OUT, 2), jnp.float32),
        ],
        scratch_shapes=[pltpu.VMEM((OUT, N2), jnp.float32)],
    )(xyz1a, xyz2a, feats1, feats2, Wi, Wf)

    out = pl.pallas_call(
        functools.partial(_bn_kernel, count=float(B * N1)),
        grid=(B, NT),
        in_specs=[
            pl.BlockSpec((1, OUT, TN), lambda b, t: (b, 0, t)),
            pl.BlockSpec((OUT, 2), lambda b, t: (0, 0)),
            pl.BlockSpec((OUT, 1), lambda b, t: (0, 0)),
            pl.BlockSpec((OUT, 1), lambda b, t: (0, 0)),
        ],
        out_specs=pl.BlockSpec((1, OUT, TN), lambda b, t: (b, 0, t)),
        out_shape=jax.ShapeDtypeStruct((B, OUT, N1), jnp.float32),
    )(y, stats, gamma.reshape(OUT, 1), beta.reshape(OUT, 1))
    return out
